# baseline (speedup 1.0000x reference)
import functools

import jax
import jax.numpy as jnp
from jax.experimental import pallas as pl
from jax.experimental.pallas import tpu as pltpu

_BLOCK = 2048
_E = 8
_BIG_I = 127


def _top2(probs, iota):
    m1 = jnp.max(probs, axis=-1, keepdims=True)
    i1 = jnp.min(jnp.where(probs == m1, iota, _BIG_I), axis=-1, keepdims=True)
    masked = jnp.where(iota == i1, -jnp.inf, probs)
    m2 = jnp.max(masked, axis=-1, keepdims=True)
    i2 = jnp.min(jnp.where(masked == m2, iota, _BIG_I), axis=-1, keepdims=True)
    return m1, i1, m2, i2


def _gate_kernel(rgb_ref, ir_ref, wt_rgb_ref, b_rgb_ref, wt_ir_ref, b_ir_ref,
                 probs_ref, idx_rgb_ref, idx_ir_ref):
    iota = jax.lax.broadcasted_iota(jnp.int32, (rgb_ref.shape[0], _E), 1)

    def gate(x, wt, b):
        s = jnp.dot(x, wt, preferred_element_type=jnp.float32) + b[0, :]
        s = s - jnp.max(s, axis=-1, keepdims=True)
        e = jnp.exp(s)
        p = e / jnp.sum(e, axis=-1, keepdims=True)
        return _top2(p, iota)

    m1r, i1r, m2r, i2r = gate(rgb_ref[...], wt_rgb_ref[...], b_rgb_ref)
    m1i, i1i, m2i, i2i = gate(ir_ref[...], wt_ir_ref[...], b_ir_ref)

    mx = jnp.maximum(jnp.maximum(m1r, m2r), jnp.maximum(m1i, m2i))
    e1r, e2r = jnp.exp(m1r - mx), jnp.exp(m2r - mx)
    e1i, e2i = jnp.exp(m1i - mx), jnp.exp(m2i - mx)
    denom = e1r + e2r + e1i + e2i
    probs_ref[...] = jnp.concatenate([e1r, e2r, e1i, e2i], axis=-1) / denom
    idx_rgb_ref[...] = jnp.concatenate([i1r, i2r], axis=-1)
    idx_ir_ref[...] = jnp.concatenate([i1i, i2i], axis=-1)


@functools.partial(jax.jit, static_argnames=("interpret",))
def kernel(rgb_local, ir_local, W_rgb, b_rgb, W_ir, b_ir, interpret=False):
    n = rgb_local.shape[0]
    d = rgb_local.shape[1]
    grid = n // _BLOCK
    wt_rgb = W_rgb.T
    wt_ir = W_ir.T
    b_rgb2 = b_rgb.reshape(1, _E)
    b_ir2 = b_ir.reshape(1, _E)

    row_spec = pl.BlockSpec((_BLOCK, d), lambda i: (i, 0))
    w_spec = pl.BlockSpec((d, _E), lambda i: (0, 0))
    b_spec = pl.BlockSpec((1, _E), lambda i: (0, 0))

    return pl.pallas_call(
        _gate_kernel,
        grid=(grid,),
        in_specs=[row_spec, row_spec, w_spec, b_spec, w_spec, b_spec],
        out_specs=[
            pl.BlockSpec((_BLOCK, 4), lambda i: (i, 0)),
            pl.BlockSpec((_BLOCK, 2), lambda i: (i, 0)),
            pl.BlockSpec((_BLOCK, 2), lambda i: (i, 0)),
        ],
        out_shape=[
            jax.ShapeDtypeStruct((n, 4), jnp.float32),
            jax.ShapeDtypeStruct((n, 2), jnp.int32),
            jax.ShapeDtypeStruct((n, 2), jnp.int32),
        ],
        interpret=interpret,
    )(rgb_local, ir_local, wt_rgb, b_rgb2, wt_ir, b_ir2)


# speedup vs baseline: 2.0569x; 2.0569x over previous
import functools

import jax
import jax.numpy as jnp
from jax.experimental import pallas as pl

_BLOCK = 2048
_E = 8
_BIG_I = 127


def _route(x, wt, b):
    s = jnp.dot(x, wt, preferred_element_type=jnp.float32)
    st = s.T + b
    iota = jax.lax.broadcasted_iota(jnp.int32, st.shape, 0)
    m1 = jnp.max(st, axis=0, keepdims=True)
    i1 = jnp.min(jnp.where(st == m1, iota, _BIG_I), axis=0, keepdims=True)
    masked = jnp.where(iota == i1, -jnp.inf, st)
    m2 = jnp.max(masked, axis=0, keepdims=True)
    i2 = jnp.min(jnp.where(masked == m2, iota, _BIG_I), axis=0, keepdims=True)
    rz = 1.0 / jnp.sum(jnp.exp(st - m1), axis=0, keepdims=True)
    return rz, jnp.exp(m2 - m1) * rz, i1, i2


def _gate_kernel(rgb_ref, ir_ref, wt_rgb_ref, b_rgb_ref, wt_ir_ref, b_ir_ref,
                 probs_ref, idx_rgb_ref, idx_ir_ref):
    p1r, p2r, i1r, i2r = _route(rgb_ref[...], wt_rgb_ref[...], b_rgb_ref[...])
    p1i, p2i, i1i, i2i = _route(ir_ref[...], wt_ir_ref[...], b_ir_ref[...])

    e1r, e2r = jnp.exp(p1r), jnp.exp(p2r)
    e1i, e2i = jnp.exp(p1i), jnp.exp(p2i)
    rden = 1.0 / (e1r + e2r + e1i + e2i)
    probs_ref[...] = jnp.concatenate([e1r, e2r, e1i, e2i], axis=0) * rden
    idx_rgb_ref[...] = jnp.concatenate([i1r, i2r], axis=0)
    idx_ir_ref[...] = jnp.concatenate([i1i, i2i], axis=0)


@functools.partial(jax.jit, static_argnames=("interpret",))
def kernel(rgb_local, ir_local, W_rgb, b_rgb, W_ir, b_ir, interpret=False):
    n = rgb_local.shape[0]
    d = rgb_local.shape[1]
    grid = n // _BLOCK

    row_spec = pl.BlockSpec((_BLOCK, d), lambda i: (i, 0))
    w_spec = pl.BlockSpec((d, _E), lambda i: (0, 0))
    b_spec = pl.BlockSpec((_E, 1), lambda i: (0, 0))

    probs_t, idx_rgb_t, idx_ir_t = pl.pallas_call(
        _gate_kernel,
        grid=(grid,),
        in_specs=[row_spec, row_spec, w_spec, b_spec, w_spec, b_spec],
        out_specs=[
            pl.BlockSpec((4, _BLOCK), lambda i: (0, i)),
            pl.BlockSpec((2, _BLOCK), lambda i: (0, i)),
            pl.BlockSpec((2, _BLOCK), lambda i: (0, i)),
        ],
        out_shape=[
            jax.ShapeDtypeStruct((4, n), jnp.float32),
            jax.ShapeDtypeStruct((2, n), jnp.int32),
            jax.ShapeDtypeStruct((2, n), jnp.int32),
        ],
        interpret=interpret,
    )(rgb_local, ir_local, W_rgb.T, b_rgb.reshape(_E, 1),
      W_ir.T, b_ir.reshape(_E, 1))
    return probs_t.T, idx_rgb_t.T, idx_ir_t.T
